# baseline (speedup 1.0000x reference)
import functools

import jax
import jax.numpy as jnp
from jax import lax
from jax.experimental import pallas as pl
from jax.experimental.pallas import tpu as pltpu
from jax.experimental.pallas import tpu_sc as plsc

KC = 1024
DD = 256
NTOK = 32 * 1024

TOK_BLK = 1024
NB = NTOK // TOK_BLK

NC, NS = 2, 16
NW = NC * NS
CH = 128
NCH = NTOK // NW // CH


def _argmin_body(xsq_ref, wsq_ref, rk_ref, x_ref, w_ref, idx_ref):
    x = x_ref[...]
    w = w_ref[...]
    mm2 = lax.dot_general(w + w, x, (((1,), (1,)), ((), ())),
                          preferred_element_type=jnp.float32)
    d = (xsq_ref[...].reshape(1, TOK_BLK) + wsq_ref[...]) - mm2
    m = jnp.min(d, axis=0, keepdims=True)
    r = jnp.max(jnp.where(d == m, rk_ref[...], -1.0), axis=0, keepdims=True)
    idx_ref[...] = (jnp.float32(KC - 1) - r).astype(jnp.int32).reshape(1, 1, TOK_BLK)


NSLAB = 4
SLAB = NTOK // NSLAB
NB_S = SLAB // TOK_BLK
NCH_S = SLAB // NW // CH


_argmin_call = pl.pallas_call(
    _argmin_body,
    grid=(NB_S,),
    in_specs=[
        pl.BlockSpec((1, 1, TOK_BLK), lambda i: (i, 0, 0)),
        pl.BlockSpec((KC, 1), lambda i: (0, 0)),
        pl.BlockSpec((KC, 1), lambda i: (0, 0)),
        pl.BlockSpec((TOK_BLK, DD), lambda i: (i, 0)),
        pl.BlockSpec((KC, DD), lambda i: (0, 0)),
    ],
    out_specs=pl.BlockSpec((1, 1, TOK_BLK), lambda i: (i, 0, 0)),
    out_shape=jax.ShapeDtypeStruct((NB_S, 1, TOK_BLK), jnp.int32),
)


@functools.lru_cache(maxsize=None)
def _make_gather():
    @functools.partial(
        pl.kernel,
        out_type=jax.ShapeDtypeStruct((SLAB, DD), jnp.float32),
        mesh=plsc.VectorSubcoreMesh(core_axis_name="c", subcore_axis_name="s"),
        scratch_types=[
            pltpu.VMEM((NCH_S, CH), jnp.int32),
            pltpu.VMEM((3, CH, DD), jnp.float32),
            pltpu.SemaphoreType.DMA,
            pltpu.SemaphoreType.DMA,
            pltpu.SemaphoreType.DMA,
            pltpu.SemaphoreType.DMA,
            pltpu.SemaphoreType.DMA,
            pltpu.SemaphoreType.DMA,
        ],
    )
    def _gather_call(idx_hbm, w_hbm, out_hbm, idx_v, ring,
                     g0, g1, g2, w0, w1, w2):
        wid = lax.axis_index("s") * NC + lax.axis_index("c")
        base = wid * (NCH_S * CH)
        gsems = (g0, g1, g2)
        wsems = (w0, w1, w2)
        pltpu.sync_copy(idx_hbm.at[wid], idx_v)
        gets = [None] * NCH_S
        puts = [None] * 3
        gets[0] = pltpu.async_copy(w_hbm.at[idx_v.at[0]], ring.at[0], gsems[0])
        if NCH_S > 1:
            gets[1] = pltpu.async_copy(w_hbm.at[idx_v.at[1]], ring.at[1], gsems[1])
        for c in range(NCH_S):
            b = c % 3
            gets[c].wait()
            puts[b] = pltpu.async_copy(ring.at[b], out_hbm.at[pl.ds(base + c * CH, CH)],
                                       wsems[b])
            if c + 2 < NCH_S:
                bn = (c + 2) % 3
                if puts[bn] is not None:
                    puts[bn].wait()
                    puts[bn] = None
                gets[c + 2] = pltpu.async_copy(w_hbm.at[idx_v.at[c + 2]], ring.at[bn],
                                               gsems[bn])
        for b in range(3):
            if puts[b] is not None:
                puts[b].wait()

    return _gather_call


def kernel(latents, W):
    shape = latents.shape
    flat = latents.reshape(-1, W.shape[1])
    xsq = jnp.sum(flat ** 2, axis=1).reshape(NSLAB, NB_S, 1, TOK_BLK)
    wsq = jnp.sum(W ** 2, axis=1)[:, None]
    rk = (jnp.float32(KC - 1) - lax.iota(jnp.float32, KC))[:, None]
    gather = _make_gather()
    idxs = [_argmin_call(xsq[s], wsq, rk, flat[s * SLAB:(s + 1) * SLAB], W)
            for s in range(NSLAB)]
    slabs = [gather(idx.reshape(NW, NCH_S, CH), W) for idx in idxs]
    return jnp.concatenate(slabs, axis=0).reshape(shape)


# speedup vs baseline: 1.3807x; 1.3807x over previous
import functools

import jax
import jax.numpy as jnp
from jax import lax
from jax.experimental import pallas as pl
from jax.experimental.pallas import tpu as pltpu
from jax.experimental.pallas import tpu_sc as plsc

KC = 1024
DD = 256
NTOK = 32 * 1024

TOK_BLK = 1024
NB = NTOK // TOK_BLK

NC, NS = 2, 16
NW = NC * NS
CH = 128
NCH = NTOK // NW // CH


def _argmin_body(xsq_ref, wsq_ref, rk_ref, x_ref, w_ref, idx_ref):
    x = x_ref[...]
    w = w_ref[...]
    mm2 = lax.dot_general(w + w, x, (((1,), (1,)), ((), ())),
                          preferred_element_type=jnp.float32)
    d = (xsq_ref[...].reshape(1, TOK_BLK) + wsq_ref[...]) - mm2
    m = jnp.min(d, axis=0, keepdims=True)
    r = jnp.max(jnp.where(d == m, rk_ref[...], -1.0), axis=0, keepdims=True)
    idx_ref[...] = (jnp.float32(KC - 1) - r).astype(jnp.int32).reshape(1, 1, TOK_BLK)


NSLAB = 1
SLAB = NTOK // NSLAB
NB_S = SLAB // TOK_BLK
NCH_S = SLAB // NW // CH


_argmin_call = pl.pallas_call(
    _argmin_body,
    grid=(NB_S,),
    in_specs=[
        pl.BlockSpec((1, 1, TOK_BLK), lambda i: (i, 0, 0)),
        pl.BlockSpec((KC, 1), lambda i: (0, 0)),
        pl.BlockSpec((KC, 1), lambda i: (0, 0)),
        pl.BlockSpec((TOK_BLK, DD), lambda i: (i, 0)),
        pl.BlockSpec((KC, DD), lambda i: (0, 0)),
    ],
    out_specs=pl.BlockSpec((1, 1, TOK_BLK), lambda i: (i, 0, 0)),
    out_shape=jax.ShapeDtypeStruct((NB_S, 1, TOK_BLK), jnp.int32),
)


@functools.lru_cache(maxsize=None)
def _make_gather():
    @functools.partial(
        pl.kernel,
        out_type=jax.ShapeDtypeStruct((SLAB, DD), jnp.float32),
        mesh=plsc.VectorSubcoreMesh(core_axis_name="c", subcore_axis_name="s"),
        scratch_types=[
            pltpu.VMEM((NCH_S, CH), jnp.int32),
            pltpu.VMEM((3, CH, DD), jnp.float32),
            pltpu.SemaphoreType.DMA,
            pltpu.SemaphoreType.DMA,
            pltpu.SemaphoreType.DMA,
            pltpu.SemaphoreType.DMA,
            pltpu.SemaphoreType.DMA,
            pltpu.SemaphoreType.DMA,
        ],
    )
    def _gather_call(idx_hbm, w_hbm, out_hbm, idx_v, ring,
                     g0, g1, g2, w0, w1, w2):
        wid = lax.axis_index("s") * NC + lax.axis_index("c")
        base = wid * (NCH_S * CH)
        gsems = (g0, g1, g2)
        wsems = (w0, w1, w2)
        pltpu.sync_copy(idx_hbm.at[wid], idx_v)
        gets = [None] * NCH_S
        puts = [None] * 3
        gets[0] = pltpu.async_copy(w_hbm.at[idx_v.at[0]], ring.at[0], gsems[0])
        if NCH_S > 1:
            gets[1] = pltpu.async_copy(w_hbm.at[idx_v.at[1]], ring.at[1], gsems[1])
        for c in range(NCH_S):
            b = c % 3
            gets[c].wait()
            puts[b] = pltpu.async_copy(ring.at[b], out_hbm.at[pl.ds(base + c * CH, CH)],
                                       wsems[b])
            if c + 2 < NCH_S:
                bn = (c + 2) % 3
                if puts[bn] is not None:
                    puts[bn].wait()
                    puts[bn] = None
                gets[c + 2] = pltpu.async_copy(w_hbm.at[idx_v.at[c + 2]], ring.at[bn],
                                               gsems[bn])
        for b in range(3):
            if puts[b] is not None:
                puts[b].wait()

    return _gather_call


def kernel(latents, W):
    shape = latents.shape
    flat = latents.reshape(-1, W.shape[1])
    xsq = jnp.sum(flat ** 2, axis=1).reshape(NSLAB, NB_S, 1, TOK_BLK)
    wsq = jnp.sum(W ** 2, axis=1)[:, None]
    rk = (jnp.float32(KC - 1) - lax.iota(jnp.float32, KC))[:, None]
    gather = _make_gather()
    idxs = [_argmin_call(xsq[s], wsq, rk, flat[s * SLAB:(s + 1) * SLAB], W)
            for s in range(NSLAB)]
    slabs = [gather(idx.reshape(NW, NCH_S, CH), W) for idx in idxs]
    return jnp.concatenate(slabs, axis=0).reshape(shape)


# speedup vs baseline: 1.4301x; 1.0358x over previous
import functools

import jax
import jax.numpy as jnp
from jax import lax
from jax.experimental import pallas as pl
from jax.experimental.pallas import tpu as pltpu
from jax.experimental.pallas import tpu_sc as plsc

KC = 1024
DD = 256
NTOK = 32 * 1024

TOK_BLK = 2048
NB = NTOK // TOK_BLK

NC, NS = 2, 16
NW = NC * NS
CH = 128
NCH = NTOK // NW // CH


def _argmin_body(xsq_ref, wsq_ref, rk_ref, x_ref, w_ref, idx_ref):
    x = x_ref[...]
    w = w_ref[...]
    mm2 = lax.dot_general(w + w, x, (((1,), (1,)), ((), ())),
                          preferred_element_type=jnp.float32)
    d = (xsq_ref[...].reshape(1, TOK_BLK) + wsq_ref[...]) - mm2
    m = jnp.min(d, axis=0, keepdims=True)
    r = jnp.max(jnp.where(d == m, rk_ref[...], -1.0), axis=0, keepdims=True)
    idx_ref[...] = (jnp.float32(KC - 1) - r).astype(jnp.int32).reshape(1, 1, TOK_BLK)


NSLAB = 1
SLAB = NTOK // NSLAB
NB_S = SLAB // TOK_BLK
NCH_S = SLAB // NW // CH


_argmin_call = pl.pallas_call(
    _argmin_body,
    grid=(NB_S,),
    in_specs=[
        pl.BlockSpec((1, 1, TOK_BLK), lambda i: (i, 0, 0)),
        pl.BlockSpec((KC, 1), lambda i: (0, 0)),
        pl.BlockSpec((KC, 1), lambda i: (0, 0)),
        pl.BlockSpec((TOK_BLK, DD), lambda i: (i, 0)),
        pl.BlockSpec((KC, DD), lambda i: (0, 0)),
    ],
    out_specs=pl.BlockSpec((1, 1, TOK_BLK), lambda i: (i, 0, 0)),
    out_shape=jax.ShapeDtypeStruct((NB_S, 1, TOK_BLK), jnp.int32),
)


@functools.lru_cache(maxsize=None)
def _make_gather():
    @functools.partial(
        pl.kernel,
        out_type=jax.ShapeDtypeStruct((SLAB, DD), jnp.float32),
        mesh=plsc.VectorSubcoreMesh(core_axis_name="c", subcore_axis_name="s"),
        scratch_types=[
            pltpu.VMEM((NCH_S, CH), jnp.int32),
            pltpu.VMEM((3, CH, DD), jnp.float32),
            pltpu.SemaphoreType.DMA,
            pltpu.SemaphoreType.DMA,
            pltpu.SemaphoreType.DMA,
            pltpu.SemaphoreType.DMA,
            pltpu.SemaphoreType.DMA,
            pltpu.SemaphoreType.DMA,
        ],
    )
    def _gather_call(idx_hbm, w_hbm, out_hbm, idx_v, ring,
                     g0, g1, g2, w0, w1, w2):
        wid = lax.axis_index("s") * NC + lax.axis_index("c")
        base = wid * (NCH_S * CH)
        gsems = (g0, g1, g2)
        wsems = (w0, w1, w2)
        pltpu.sync_copy(idx_hbm.at[wid], idx_v)
        gets = [None] * NCH_S
        puts = [None] * 3
        gets[0] = pltpu.async_copy(w_hbm.at[idx_v.at[0]], ring.at[0], gsems[0])
        if NCH_S > 1:
            gets[1] = pltpu.async_copy(w_hbm.at[idx_v.at[1]], ring.at[1], gsems[1])
        for c in range(NCH_S):
            b = c % 3
            gets[c].wait()
            puts[b] = pltpu.async_copy(ring.at[b], out_hbm.at[pl.ds(base + c * CH, CH)],
                                       wsems[b])
            if c + 2 < NCH_S:
                bn = (c + 2) % 3
                if puts[bn] is not None:
                    puts[bn].wait()
                    puts[bn] = None
                gets[c + 2] = pltpu.async_copy(w_hbm.at[idx_v.at[c + 2]], ring.at[bn],
                                               gsems[bn])
        for b in range(3):
            if puts[b] is not None:
                puts[b].wait()

    return _gather_call


def kernel(latents, W):
    shape = latents.shape
    flat = latents.reshape(-1, W.shape[1])
    xsq = jnp.sum(flat ** 2, axis=1).reshape(NSLAB, NB_S, 1, TOK_BLK)
    wsq = jnp.sum(W ** 2, axis=1)[:, None]
    rk = (jnp.float32(KC - 1) - lax.iota(jnp.float32, KC))[:, None]
    gather = _make_gather()
    idxs = [_argmin_call(xsq[s], wsq, rk, flat[s * SLAB:(s + 1) * SLAB], W)
            for s in range(NSLAB)]
    slabs = [gather(idx.reshape(NW, NCH_S, CH), W) for idx in idxs]
    return jnp.concatenate(slabs, axis=0).reshape(shape)


# speedup vs baseline: 1.4502x; 1.0140x over previous
import functools

import jax
import jax.numpy as jnp
from jax import lax
from jax.experimental import pallas as pl
from jax.experimental.pallas import tpu as pltpu
from jax.experimental.pallas import tpu_sc as plsc

KC = 1024
DD = 256
NTOK = 32 * 1024

TOK_BLK = 4096
NB = NTOK // TOK_BLK

NC, NS = 2, 16
NW = NC * NS
CH = 128
NCH = NTOK // NW // CH


def _argmin_body(xsq_ref, wsq_ref, rk_ref, x_ref, w_ref, idx_ref):
    x = x_ref[...]
    w = w_ref[...]
    mm2 = lax.dot_general(w + w, x, (((1,), (1,)), ((), ())),
                          preferred_element_type=jnp.float32)
    d = (xsq_ref[...].reshape(1, TOK_BLK) + wsq_ref[...]) - mm2
    m = jnp.min(d, axis=0, keepdims=True)
    r = jnp.max(jnp.where(d == m, rk_ref[...], -1.0), axis=0, keepdims=True)
    idx_ref[...] = (jnp.float32(KC - 1) - r).astype(jnp.int32).reshape(1, 1, TOK_BLK)


NSLAB = 1
SLAB = NTOK // NSLAB
NB_S = SLAB // TOK_BLK
NCH_S = SLAB // NW // CH


_argmin_call = pl.pallas_call(
    _argmin_body,
    grid=(NB_S,),
    in_specs=[
        pl.BlockSpec((1, 1, TOK_BLK), lambda i: (i, 0, 0)),
        pl.BlockSpec((KC, 1), lambda i: (0, 0)),
        pl.BlockSpec((KC, 1), lambda i: (0, 0)),
        pl.BlockSpec((TOK_BLK, DD), lambda i: (i, 0)),
        pl.BlockSpec((KC, DD), lambda i: (0, 0)),
    ],
    out_specs=pl.BlockSpec((1, 1, TOK_BLK), lambda i: (i, 0, 0)),
    out_shape=jax.ShapeDtypeStruct((NB_S, 1, TOK_BLK), jnp.int32),
)


@functools.lru_cache(maxsize=None)
def _make_gather():
    @functools.partial(
        pl.kernel,
        out_type=jax.ShapeDtypeStruct((SLAB, DD), jnp.float32),
        mesh=plsc.VectorSubcoreMesh(core_axis_name="c", subcore_axis_name="s"),
        scratch_types=[
            pltpu.VMEM((NCH_S, CH), jnp.int32),
            pltpu.VMEM((3, CH, DD), jnp.float32),
            pltpu.SemaphoreType.DMA,
            pltpu.SemaphoreType.DMA,
            pltpu.SemaphoreType.DMA,
            pltpu.SemaphoreType.DMA,
            pltpu.SemaphoreType.DMA,
            pltpu.SemaphoreType.DMA,
        ],
    )
    def _gather_call(idx_hbm, w_hbm, out_hbm, idx_v, ring,
                     g0, g1, g2, w0, w1, w2):
        wid = lax.axis_index("s") * NC + lax.axis_index("c")
        base = wid * (NCH_S * CH)
        gsems = (g0, g1, g2)
        wsems = (w0, w1, w2)
        pltpu.sync_copy(idx_hbm.at[wid], idx_v)
        gets = [None] * NCH_S
        puts = [None] * 3
        gets[0] = pltpu.async_copy(w_hbm.at[idx_v.at[0]], ring.at[0], gsems[0])
        if NCH_S > 1:
            gets[1] = pltpu.async_copy(w_hbm.at[idx_v.at[1]], ring.at[1], gsems[1])
        for c in range(NCH_S):
            b = c % 3
            gets[c].wait()
            puts[b] = pltpu.async_copy(ring.at[b], out_hbm.at[pl.ds(base + c * CH, CH)],
                                       wsems[b])
            if c + 2 < NCH_S:
                bn = (c + 2) % 3
                if puts[bn] is not None:
                    puts[bn].wait()
                    puts[bn] = None
                gets[c + 2] = pltpu.async_copy(w_hbm.at[idx_v.at[c + 2]], ring.at[bn],
                                               gsems[bn])
        for b in range(3):
            if puts[b] is not None:
                puts[b].wait()

    return _gather_call


def kernel(latents, W):
    shape = latents.shape
    flat = latents.reshape(-1, W.shape[1])
    xsq = jnp.sum(flat ** 2, axis=1).reshape(NSLAB, NB_S, 1, TOK_BLK)
    wsq = jnp.sum(W ** 2, axis=1)[:, None]
    rk = (jnp.float32(KC - 1) - lax.iota(jnp.float32, KC))[:, None]
    gather = _make_gather()
    idxs = [_argmin_call(xsq[s], wsq, rk, flat[s * SLAB:(s + 1) * SLAB], W)
            for s in range(NSLAB)]
    slabs = [gather(idx.reshape(NW, NCH_S, CH), W) for idx in idxs]
    return jnp.concatenate(slabs, axis=0).reshape(shape)
